# baseline (speedup 1.0000x reference)
import functools

import jax
import jax.numpy as jnp
from jax.experimental import pallas as pl

_BN = 256
_HID = 128


def _proj_body(x_ref, w_ref, att_ref, h_ref, a_ref):
    h = jnp.dot(x_ref[...], w_ref[...], preferred_element_type=jnp.float32)
    h_ref[...] = h
    a_ref[...] = jnp.dot(h, att_ref[...], preferred_element_type=jnp.float32)


@functools.partial(jax.jit, static_argnames=())
def _proj(x, W, att):
    n = x.shape[0]
    npad = ((n + _BN - 1) // _BN) * _BN
    xp = jnp.pad(x, ((0, npad - n), (0, 0)))
    h, a = pl.pallas_call(
        _proj_body,
        grid=(npad // _BN,),
        in_specs=[
            pl.BlockSpec((_BN, _HID), lambda i: (i, 0)),
            pl.BlockSpec((_HID, _HID), lambda i: (0, 0)),
            pl.BlockSpec((_HID, 1), lambda i: (0, 0)),
        ],
        out_specs=[
            pl.BlockSpec((_BN, _HID), lambda i: (i, 0)),
            pl.BlockSpec((_BN, 1), lambda i: (i, 0)),
        ],
        out_shape=[
            jax.ShapeDtypeStruct((npad, _HID), jnp.float32),
            jax.ShapeDtypeStruct((npad, 1), jnp.float32),
        ],
    )(xp, W, att[:, None])
    return h[:n], a[:n, 0]


def _gat(x_src, x_dst, edge_index, p):
    n_src, n_dst = x_src.shape[0], x_dst.shape[0]
    src, dst = edge_index[0], edge_index[1]
    keep = src != dst
    n_loop = min(n_src, n_dst)
    loop = jnp.arange(n_loop, dtype=src.dtype)
    src = jnp.concatenate([src, loop])
    dst = jnp.concatenate([dst, loop])
    keep = jnp.concatenate([keep, jnp.ones((n_loop,), jnp.bool_)])

    hs, a_src = _proj(x_src, p['W_src'], p['att_src'])
    _, a_dst = _proj(x_dst, p['W_dst'], p['att_dst'])

    logits = jax.nn.leaky_relu(a_src[src] + a_dst[dst], 0.2)
    logits = jnp.where(keep, logits, -jnp.inf)
    m = jax.ops.segment_max(logits, dst, num_segments=n_dst)
    m = jnp.where(jnp.isfinite(m), m, 0.0)
    e = jnp.where(keep, jnp.exp(logits - m[dst]), 0.0)
    denom = jax.ops.segment_sum(e, dst, num_segments=n_dst)
    coef = e / jnp.maximum(denom[dst], 1e-16)
    out = jax.ops.segment_sum(coef[:, None] * hs[src], dst, num_segments=n_dst)
    return out + p['bias']


def kernel(x_concept, x_entity, x_relation, x_event, type_emb, params,
           ei_has, ei_rel, ei_inv, ei_bel):
    x = {
        'concept': x_concept + type_emb[0],
        'entity': x_entity + type_emb[1],
        'event': x_event + type_emb[3],
    }
    for l in range(3):
        p = params['l%d' % l]
        ent = _gat(x['concept'], x['entity'], ei_has, p['has'])
        ent = ent + _gat(x['entity'], x['entity'], ei_rel, p['rel'])
        if 'event' in x:
            ent = ent + _gat(x['event'], x['entity'], ei_inv, p['inv'])
        con = _gat(x['entity'], x['concept'], ei_bel, p['bel'])
        x = {'entity': jax.nn.relu(ent), 'concept': jax.nn.relu(con)}
    return x['entity'], x['concept']


# speedup vs baseline: 1.8291x; 1.8291x over previous
import functools

import jax
import jax.numpy as jnp
from jax.experimental import pallas as pl

_BN = 256
_HID = 128


def _proj_body(x_ref, w_ref, att_ref, h_ref, a_ref):
    h = jnp.dot(x_ref[...], w_ref[...], preferred_element_type=jnp.float32)
    h_ref[...] = h
    a_ref[...] = jnp.dot(h, att_ref[...], preferred_element_type=jnp.float32)


@functools.partial(jax.jit, static_argnames=())
def _proj(x, W, att):
    n = x.shape[0]
    npad = ((n + _BN - 1) // _BN) * _BN
    xp = jnp.pad(x, ((0, npad - n), (0, 0)))
    h, a = pl.pallas_call(
        _proj_body,
        grid=(npad // _BN,),
        in_specs=[
            pl.BlockSpec((_BN, _HID), lambda i: (i, 0)),
            pl.BlockSpec((_HID, _HID), lambda i: (0, 0)),
            pl.BlockSpec((_HID, 1), lambda i: (0, 0)),
        ],
        out_specs=[
            pl.BlockSpec((_BN, _HID), lambda i: (i, 0)),
            pl.BlockSpec((_BN, 1), lambda i: (i, 0)),
        ],
        out_shape=[
            jax.ShapeDtypeStruct((npad, _HID), jnp.float32),
            jax.ShapeDtypeStruct((npad, 1), jnp.float32),
        ],
    )(xp, W, att[:, None])
    return h[:n], a[:n, 0]


def _gat(x_src, x_dst, edge_index, p):
    n_src, n_dst = x_src.shape[0], x_dst.shape[0]
    src, dst = edge_index[0], edge_index[1]
    keep = src != dst
    n_loop = min(n_src, n_dst)
    loop = jnp.arange(n_loop, dtype=src.dtype)
    src = jnp.concatenate([src, loop])
    dst = jnp.concatenate([dst, loop])
    keep = jnp.concatenate([keep, jnp.ones((n_loop,), jnp.bool_)])

    hs, a_src = _proj(x_src, p['W_src'], p['att_src'])
    _, a_dst = _proj(x_dst, p['W_dst'], p['att_dst'])

    C = jnp.maximum(jnp.max(a_src) + jnp.max(a_dst), 0.0)
    logits = jax.nn.leaky_relu(a_src[src] + a_dst[dst], 0.2)
    e = jnp.where(keep, jnp.exp(logits - C), 0.0)
    hs1 = jnp.concatenate([hs, jnp.ones((n_src, 1), jnp.float32)], axis=1)
    num = jax.ops.segment_sum(e[:, None] * hs1[src], dst, num_segments=n_dst)
    out = num[:, :_HID] / jnp.maximum(num[:, _HID:], 1e-16)
    return out + p['bias']


def kernel(x_concept, x_entity, x_relation, x_event, type_emb, params,
           ei_has, ei_rel, ei_inv, ei_bel):
    x = {
        'concept': x_concept + type_emb[0],
        'entity': x_entity + type_emb[1],
        'event': x_event + type_emb[3],
    }
    for l in range(3):
        p = params['l%d' % l]
        ent = _gat(x['concept'], x['entity'], ei_has, p['has'])
        ent = ent + _gat(x['entity'], x['entity'], ei_rel, p['rel'])
        if 'event' in x:
            ent = ent + _gat(x['event'], x['entity'], ei_inv, p['inv'])
        con = _gat(x['entity'], x['concept'], ei_bel, p['bel'])
        x = {'entity': jax.nn.relu(ent), 'concept': jax.nn.relu(con)}
    return x['entity'], x['concept']


# speedup vs baseline: 4.0674x; 2.2238x over previous
import functools

import jax
import jax.numpy as jnp
from jax import lax
from jax.experimental import pallas as pl
from jax.experimental.pallas import tpu as pltpu, tpu_sc as plsc

_NW = 32
_L = 16

_BN = 256
_HID = 128


def _proj_body(x_ref, w_ref, att_ref, h_ref, a_ref):
    h = jnp.dot(x_ref[...], w_ref[...], preferred_element_type=jnp.float32)
    h_ref[...] = h
    a_ref[...] = jnp.dot(h, att_ref[...], preferred_element_type=jnp.float32)


@functools.partial(jax.jit, static_argnames=())
def _proj(x, W, att):
    n = x.shape[0]
    npad = ((n + _BN - 1) // _BN) * _BN
    xp = jnp.pad(x, ((0, npad - n), (0, 0)))
    h, a = pl.pallas_call(
        _proj_body,
        grid=(npad // _BN,),
        in_specs=[
            pl.BlockSpec((_BN, _HID), lambda i: (i, 0)),
            pl.BlockSpec((_HID, _HID), lambda i: (0, 0)),
            pl.BlockSpec((_HID, 1), lambda i: (0, 0)),
        ],
        out_specs=[
            pl.BlockSpec((_BN, _HID), lambda i: (i, 0)),
            pl.BlockSpec((_BN, 1), lambda i: (i, 0)),
        ],
        out_shape=[
            jax.ShapeDtypeStruct((npad, _HID), jnp.float32),
            jax.ShapeDtypeStruct((npad, 1), jnp.float32),
        ],
    )(xp, W, att[:, None])
    return h[:n], a[:n, 0]


@functools.lru_cache(maxsize=None)
def _edge_e_kernel(ns, nd, ep):
    epw = ep // _NW
    mesh = plsc.VectorSubcoreMesh(core_axis_name="c", subcore_axis_name="s")

    @functools.partial(
        pl.kernel, mesh=mesh,
        out_type=jax.ShapeDtypeStruct((ep,), jnp.float32),
        compiler_params=pltpu.CompilerParams(needs_layout_passes=False),
        scratch_types=[
            pltpu.VMEM((ns,), jnp.float32),
            pltpu.VMEM((nd,), jnp.float32),
            pltpu.VMEM((epw,), jnp.int32),
            pltpu.VMEM((epw,), jnp.int32),
            pltpu.VMEM((epw,), jnp.float32),
            pltpu.VMEM((epw,), jnp.float32),
            pltpu.VMEM((_L,), jnp.float32),
        ],
    )
    def k(asrc_h, adst_h, src_h, dst_h, keep_h, c_h, out_h,
          asrc_v, adst_v, src_v, dst_v, keep_v, e_v, c_v):
        wid = lax.axis_index("s") * 2 + lax.axis_index("c")
        base = wid * epw
        pltpu.sync_copy(asrc_h, asrc_v)
        pltpu.sync_copy(adst_h, adst_v)
        pltpu.sync_copy(src_h.at[pl.ds(base, epw)], src_v)
        pltpu.sync_copy(dst_h.at[pl.ds(base, epw)], dst_v)
        pltpu.sync_copy(keep_h.at[pl.ds(base, epw)], keep_v)
        pltpu.sync_copy(c_h, c_v)
        c = c_v[...]

        def body(i, carry):
            sl = pl.ds(i * _L, _L)
            sv = src_v[sl]
            dv = dst_v[sl]
            kv = keep_v[sl]
            ga = plsc.load_gather(asrc_v, [sv])
            gd = plsc.load_gather(adst_v, [dv])
            z = ga + gd
            lg = jnp.where(z > 0, z, 0.2 * z)
            e_v[sl] = kv * jnp.exp(lg - c)
            return carry

        lax.fori_loop(0, epw // _L, body, 0)
        pltpu.sync_copy(e_v, out_h.at[pl.ds(base, epw)])

    return k


def _edge_e(a_src, a_dst, src, dst, keep, C):
    e2 = src.shape[0]
    ep = ((e2 + 512 - 1) // 512) * 512
    pad = ep - e2
    srcp = jnp.pad(src, (0, pad))
    dstp = jnp.pad(dst, (0, pad))
    keepp = jnp.pad(keep.astype(jnp.float32), (0, pad))
    cvec = jnp.full((_L,), C, jnp.float32)
    k = _edge_e_kernel(a_src.shape[0], a_dst.shape[0], ep)
    e = k(a_src, a_dst, srcp, dstp, keepp, cvec)
    return e[:e2]


def _gat(x_src, x_dst, edge_index, p):
    n_src, n_dst = x_src.shape[0], x_dst.shape[0]
    src, dst = edge_index[0], edge_index[1]
    keep = src != dst
    n_loop = min(n_src, n_dst)
    loop = jnp.arange(n_loop, dtype=src.dtype)
    src = jnp.concatenate([src, loop])
    dst = jnp.concatenate([dst, loop])
    keep = jnp.concatenate([keep, jnp.ones((n_loop,), jnp.bool_)])

    hs, a_src = _proj(x_src, p['W_src'], p['att_src'])
    _, a_dst = _proj(x_dst, p['W_dst'], p['att_dst'])

    C = jnp.maximum(jnp.max(a_src) + jnp.max(a_dst), 0.0)
    e = _edge_e(a_src, a_dst, src, dst, keep, C)
    hs1 = jnp.concatenate([hs, jnp.ones((n_src, 1), jnp.float32)], axis=1)
    num = jax.ops.segment_sum(e[:, None] * hs1[src], dst, num_segments=n_dst)
    out = num[:, :_HID] / jnp.maximum(num[:, _HID:], 1e-16)
    return out + p['bias']


def kernel(x_concept, x_entity, x_relation, x_event, type_emb, params,
           ei_has, ei_rel, ei_inv, ei_bel):
    x = {
        'concept': x_concept + type_emb[0],
        'entity': x_entity + type_emb[1],
        'event': x_event + type_emb[3],
    }
    for l in range(3):
        p = params['l%d' % l]
        ent = _gat(x['concept'], x['entity'], ei_has, p['has'])
        ent = ent + _gat(x['entity'], x['entity'], ei_rel, p['rel'])
        if 'event' in x:
            ent = ent + _gat(x['event'], x['entity'], ei_inv, p['inv'])
        con = _gat(x['entity'], x['concept'], ei_bel, p['bel'])
        x = {'entity': jax.nn.relu(ent), 'concept': jax.nn.relu(con)}
    return x['entity'], x['concept']
